# Initial kernel scaffold; baseline (speedup 1.0000x reference)
import jax
import jax.numpy as jnp
from jax.experimental import pallas as pl


def kernel(input_ids, table, cls_domain, cls_task, pos_encoding, gamma, beta):
    raise NotImplementedError("write your pallas kernel here")


# speedup vs baseline: 3.1984x; 3.1984x over previous
import functools

import jax
import jax.numpy as jnp
from jax import lax
from jax.experimental import pallas as pl
from jax.experimental.pallas import tpu as pltpu
from jax.experimental.pallas import tpu_sc as plsc

SEQ = 200
HID = 64
POS_LEN = SEQ + 2
NT_ROWS = 40200
CLS0 = 40000
CLS1 = 40001

NC = 2
NS = 16
NW = NC * NS
CHUNK = 128


def _ln_table_body(table_ref, pos_ref, cls_ref, gamma_ref, beta_ref, out_ref):
    s = pl.program_id(0)
    srow = jnp.minimum(s + 2, POS_LEN - 1)
    posrow = pos_ref[pl.ds(srow, 1), :]
    body = table_ref[...] + posrow
    row0 = jnp.broadcast_to(cls_ref[pl.ds(0, 1), :], (SEQ, HID))
    row1 = jnp.broadcast_to(cls_ref[pl.ds(1, 1), :], (SEQ, HID))
    rr = lax.broadcasted_iota(jnp.int32, (SEQ, HID), 0)
    clsx = jnp.where(rr == 0, row0, jnp.where(rr == 1, row1, 0.0))
    x = jnp.where(s < SEQ, body, clsx)
    mean = jnp.mean(x, axis=-1, keepdims=True)
    var = jnp.mean(jnp.square(x - mean), axis=-1, keepdims=True)
    y = (x - mean) * lax.rsqrt(var + 1e-5)
    out_ref[...] = y * gamma_ref[...] + beta_ref[...]


def _build_norm_table(table, pos2, cls_rows, gamma, beta):
    return pl.pallas_call(
        _ln_table_body,
        grid=(SEQ + 1,),
        in_specs=[
            pl.BlockSpec((SEQ, HID), lambda s: (0, 0)),
            pl.BlockSpec((POS_LEN, HID), lambda s: (0, 0)),
            pl.BlockSpec((2, HID), lambda s: (0, 0)),
            pl.BlockSpec((1, HID), lambda s: (0, 0)),
            pl.BlockSpec((1, HID), lambda s: (0, 0)),
        ],
        out_specs=pl.BlockSpec((SEQ, HID), lambda s: (s, 0)),
        out_shape=jax.ShapeDtypeStruct((NT_ROWS, HID), jnp.float32),
    )(table, pos2, cls_rows, gamma.reshape(1, HID), beta.reshape(1, HID))


def _make_sc_gather(total_rows):
    rows_per_w = total_rows // NW
    n_chunks = rows_per_w // CHUNK
    mesh = plsc.VectorSubcoreMesh(core_axis_name="c", subcore_axis_name="s")

    @functools.partial(
        pl.kernel,
        mesh=mesh,
        compiler_params=pltpu.CompilerParams(use_tc_tiling_on_sc=False),
        out_type=jax.ShapeDtypeStruct((total_rows, HID), jnp.float32),
        scratch_types=[
            pltpu.VMEM((CHUNK,), jnp.int32),
            pltpu.VMEM((CHUNK, HID), jnp.float32),
            pltpu.SemaphoreType.DMA,
        ],
    )
    def gather_kernel(nt_hbm, fi_hbm, out_hbm, idx_v, rows_v, sem):
        wid = lax.axis_index("s") * NC + lax.axis_index("c")
        w_base = wid * rows_per_w

        def body(i, carry):
            base = w_base + i * CHUNK
            pltpu.sync_copy(fi_hbm.at[pl.ds(base, CHUNK)], idx_v)
            pltpu.async_copy(nt_hbm.at[idx_v], rows_v, sem).wait()
            pltpu.sync_copy(rows_v, out_hbm.at[pl.ds(base, CHUNK)])
            return carry

        lax.fori_loop(0, n_chunks, body, 0)

    return gather_kernel


def kernel(input_ids, table, cls_domain, cls_task, pos_encoding, gamma, beta):
    b = input_ids.shape[0]
    total_rows = b * POS_LEN

    pos2 = pos_encoding.reshape(POS_LEN, HID)
    cls_rows = jnp.concatenate(
        [cls_domain.reshape(1, HID) + pos2[0:1],
         cls_task.reshape(1, HID) + pos2[1:2]], axis=0)

    nt = _build_norm_table(table, pos2, cls_rows, gamma, beta)

    ids = input_ids.astype(jnp.int32)
    fi_body = ids + jnp.arange(SEQ, dtype=jnp.int32)[None, :] * SEQ
    fi = jnp.concatenate(
        [jnp.full((b, 1), CLS0, jnp.int32),
         jnp.full((b, 1), CLS1, jnp.int32),
         fi_body], axis=1).reshape(total_rows)

    out = _make_sc_gather(total_rows)(nt, fi)
    return out.reshape(b, POS_LEN, HID)


# speedup vs baseline: 3.2359x; 1.0117x over previous
import functools

import jax
import jax.numpy as jnp
from jax import lax
from jax.experimental import pallas as pl
from jax.experimental.pallas import tpu as pltpu
from jax.experimental.pallas import tpu_sc as plsc

SEQ = 200
HID = 64
POS_LEN = SEQ + 2
NT_ROWS = 40200
CLS0 = 40000
CLS1 = 40001

NC = 2
NS = 16
NW = NC * NS
CHUNK = 128


def _ln_table_body(table_ref, pos_ref, cls_ref, gamma_ref, beta_ref, out_ref):
    s = pl.program_id(0)
    srow = jnp.minimum(s + 2, POS_LEN - 1)
    posrow = pos_ref[pl.ds(srow, 1), :]
    body = table_ref[...] + posrow
    row0 = jnp.broadcast_to(cls_ref[pl.ds(0, 1), :], (SEQ, HID))
    row1 = jnp.broadcast_to(cls_ref[pl.ds(1, 1), :], (SEQ, HID))
    rr = lax.broadcasted_iota(jnp.int32, (SEQ, HID), 0)
    clsx = jnp.where(rr == 0, row0, jnp.where(rr == 1, row1, 0.0))
    x = jnp.where(s < SEQ, body, clsx)
    mean = jnp.mean(x, axis=-1, keepdims=True)
    var = jnp.mean(jnp.square(x - mean), axis=-1, keepdims=True)
    y = (x - mean) * lax.rsqrt(var + 1e-5)
    out_ref[...] = y * gamma_ref[...] + beta_ref[...]


def _build_norm_table(table, pos2, cls_rows, gamma, beta):
    return pl.pallas_call(
        _ln_table_body,
        grid=(SEQ + 1,),
        in_specs=[
            pl.BlockSpec((SEQ, HID), lambda s: (0, 0)),
            pl.BlockSpec((POS_LEN, HID), lambda s: (0, 0)),
            pl.BlockSpec((2, HID), lambda s: (0, 0)),
            pl.BlockSpec((1, HID), lambda s: (0, 0)),
            pl.BlockSpec((1, HID), lambda s: (0, 0)),
        ],
        out_specs=pl.BlockSpec((SEQ, HID), lambda s: (s, 0)),
        out_shape=jax.ShapeDtypeStruct((NT_ROWS, HID), jnp.float32),
    )(table, pos2, cls_rows, gamma.reshape(1, HID), beta.reshape(1, HID))


CROWS = CHUNK


def _make_sc_gather(total_rows):
    rows_per_w = total_rows // NW
    n_grp = rows_per_w // CHUNK
    n_full = n_grp
    n_pair = n_full // 2
    mesh = plsc.VectorSubcoreMesh(core_axis_name="c", subcore_axis_name="s")

    @functools.partial(
        pl.kernel,
        mesh=mesh,
        compiler_params=pltpu.CompilerParams(use_tc_tiling_on_sc=False),
        out_type=jax.ShapeDtypeStruct((total_rows, HID), jnp.float32),
        scratch_types=[
            pltpu.VMEM((n_grp, CHUNK), jnp.int32),
            pltpu.VMEM((2, CROWS, HID), jnp.float32),
            pltpu.SemaphoreType.DMA,
            pltpu.SemaphoreType.DMA,
            pltpu.SemaphoreType.DMA,
            pltpu.SemaphoreType.DMA,
        ],
    )
    def gather_kernel(nt_hbm, fi_hbm, out_hbm, idx_v, rows_v, g0, g1, w0, w1):
        wid = lax.axis_index("s") * NC + lax.axis_index("c")
        w_base = wid * rows_per_w

        pltpu.sync_copy(fi_hbm.at[pl.ds(wid * n_grp, n_grp)], idx_v)

        gsem = (g0, g1)
        wsem = (w0, w1)

        def gath(c, slot):
            return pltpu.make_async_copy(
                nt_hbm.at[idx_v.at[c]], rows_v.at[slot], gsem[slot])

        def writ(c, slot):
            return pltpu.make_async_copy(
                rows_v.at[slot], out_hbm.at[pl.ds(w_base + c * CROWS, CROWS)],
                wsem[slot])

        gath(0, 0).start()

        def body(g, carry):
            a = 2 * g
            b = a + 1

            @pl.when(g > 0)
            def _():
                writ(b - 2, 1).wait()
            gath(b, 1).start()
            gath(a, 0).wait()
            writ(a, 0).start()

            @pl.when(g < n_pair - 1)
            def _():
                writ(a, 0).wait()
                gath(a + 2, 0).start()
            gath(b, 1).wait()
            writ(b, 1).start()
            return carry

        lax.fori_loop(0, n_pair, body, 0)

        writ(n_full - 2, 0).wait()
        writ(n_full - 1, 1).wait()

    return gather_kernel


def kernel(input_ids, table, cls_domain, cls_task, pos_encoding, gamma, beta):
    b = input_ids.shape[0]
    total_rows = b * POS_LEN

    pos2 = pos_encoding.reshape(POS_LEN, HID)
    cls_rows = jnp.concatenate(
        [cls_domain.reshape(1, HID) + pos2[0:1],
         cls_task.reshape(1, HID) + pos2[1:2]], axis=0)

    nt = _build_norm_table(table, pos2, cls_rows, gamma, beta)

    ids = input_ids.astype(jnp.int32)
    fi_body = ids + jnp.arange(SEQ, dtype=jnp.int32)[None, :] * SEQ
    fi = jnp.concatenate(
        [jnp.full((b, 1), CLS0, jnp.int32),
         jnp.full((b, 1), CLS1, jnp.int32),
         fi_body], axis=1).reshape(total_rows // CHUNK, CHUNK)

    out = _make_sc_gather(total_rows)(nt, fi)
    return out.reshape(b, POS_LEN, HID)
